# initial kernel scaffold (unmeasured)
import jax
import jax.numpy as jnp
from jax import lax
from jax.experimental import pallas as pl
from jax.experimental.pallas import tpu as pltpu

N_DEV = 8
BLK = 64


def kernel(x, Wq, K_ext, V_ext, Wo):
    B, Sq_l, D = x.shape
    _, Skv_l, Hq, Dh = K_ext.shape
    HD = Hq * Dh

    k2 = K_ext.reshape(B, Skv_l, HD)
    v2 = V_ext.reshape(B, Skv_l, HD)

    def body(x_ref, wq_ref, k_ref, v_ref, wo_ref, out_ref,
             kbuf, vbuf, send_sems, recv_sems):
        my = lax.axis_index("i")
        left = lax.rem(my + N_DEV - 1, N_DEV)
        right = lax.rem(my + 1, N_DEV)

        barrier = pltpu.get_barrier_semaphore()
        for nbr in (left, right):
            pl.semaphore_signal(barrier, inc=1, device_id=(nbr,),
                                device_id_type=pl.DeviceIdType.MESH)
        pl.semaphore_wait(barrier, 2)

        kbuf[my] = k_ref[...]
        vbuf[my] = v_ref[...]

        for h in range(N_DEV - 1):
            src = lax.rem(my + N_DEV - h, N_DEV)
            rk = pltpu.make_async_remote_copy(
                src_ref=kbuf.at[src], dst_ref=kbuf.at[src],
                send_sem=send_sems.at[0, h], recv_sem=recv_sems.at[0, h],
                device_id=(right,), device_id_type=pl.DeviceIdType.MESH)
            rv = pltpu.make_async_remote_copy(
                src_ref=vbuf.at[src], dst_ref=vbuf.at[src],
                send_sem=send_sems.at[1, h], recv_sem=recv_sems.at[1, h],
                device_id=(right,), device_id_type=pl.DeviceIdType.MESH)
            rk.start()
            rv.start()
            rk.wait()
            rv.wait()

        xf = x_ref[...].reshape(B * Sq_l, D)
        qf = jnp.dot(xf, wq_ref[...], preferred_element_type=jnp.float32)

        qblocks_per_shard = Sq_l // BLK
        for b in range(B):
            ctx_heads = []
            for hq in range(Hq):
                qbh = qf[b * Sq_l:(b + 1) * Sq_l, hq * Dh:(hq + 1) * Dh]
                m = jnp.full((Sq_l, 1), -1e30, jnp.float32)
                blocks = []
                for o in range(N_DEV):
                    kc = kbuf[o, b]
                    s = lax.dot_general(
                        qbh, kc[:, hq * Dh:(hq + 1) * Dh],
                        (((1,), (1,)), ((), ())),
                        preferred_element_type=jnp.float32)
                    qi = lax.broadcasted_iota(jnp.int32, (Sq_l, Skv_l), 0)
                    ki = lax.broadcasted_iota(jnp.int32, (Sq_l, Skv_l), 1)
                    qb = qi // BLK + my * qblocks_per_shard
                    kb = ki // BLK + o * (Skv_l // BLK)
                    mask = (qb == kb) | (kb == 0) | (lax.rem(qb + kb, 3) == 0)
                    s = jnp.where(mask, s * 0.125, -1e9)
                    blocks.append(s)
                    m = jnp.maximum(m, s.max(axis=1, keepdims=True))
                ctx = jnp.zeros((Sq_l, Dh), jnp.float32)
                den = jnp.zeros((Sq_l, 1), jnp.float32)
                for o in range(N_DEV):
                    e = jnp.exp(blocks[o] - m)
                    den = den + e.sum(axis=1, keepdims=True)
                    vc = vbuf[o, b]
                    ctx = ctx + jnp.dot(e, vc[:, hq * Dh:(hq + 1) * Dh],
                                        preferred_element_type=jnp.float32)
                ctx_heads.append(ctx / den)
            ctx_b = jnp.concatenate(ctx_heads, axis=1)
            out_ref[b] = jnp.dot(ctx_b, wo_ref[...],
                                 preferred_element_type=jnp.float32)

    return pl.pallas_call(
        body,
        out_shape=jax.ShapeDtypeStruct((B, Sq_l, D), jnp.float32),
        in_specs=[pl.BlockSpec(memory_space=pltpu.VMEM)] * 5,
        out_specs=pl.BlockSpec(memory_space=pltpu.VMEM),
        scratch_shapes=[
            pltpu.VMEM((N_DEV, B, Skv_l, HD), jnp.float32),
            pltpu.VMEM((N_DEV, B, Skv_l, HD), jnp.float32),
            pltpu.SemaphoreType.DMA((2, N_DEV - 1)),
            pltpu.SemaphoreType.DMA((2, N_DEV - 1)),
        ],
        compiler_params=pltpu.CompilerParams(collective_id=0),
    )(x, Wq, k2, v2)


# baseline (device time: 113577 ns/iter reference)
import jax
import jax.numpy as jnp
from jax import lax
from jax.experimental import pallas as pl
from jax.experimental.pallas import tpu as pltpu

N_DEV = 8
BLK = 64


def kernel(x, Wq, K_ext, V_ext, Wo):
    B, Sq_l, D = x.shape
    _, Skv_l, Hq, Dh = K_ext.shape
    HD = Hq * Dh

    k2 = K_ext.reshape(B, Skv_l, HD)
    v2 = V_ext.reshape(B, Skv_l, HD)

    def body(x_ref, wq_ref, k_ref, v_ref, wo_ref, out_ref,
             kbuf, vbuf, send_sems, recv_sems):
        my = lax.axis_index("i")
        left = lax.rem(my + N_DEV - 1, N_DEV)
        right = lax.rem(my + 1, N_DEV)

        barrier = pltpu.get_barrier_semaphore()
        for nbr in (left, right):
            pl.semaphore_signal(barrier, inc=1, device_id=(nbr,),
                                device_id_type=pl.DeviceIdType.MESH)
        pl.semaphore_wait(barrier, 2)

        kbuf[my] = k_ref[...]
        vbuf[my] = v_ref[...]

        for h in range(N_DEV - 1):
            src = lax.rem(my + N_DEV - h, N_DEV)
            rk = pltpu.make_async_remote_copy(
                src_ref=kbuf.at[src], dst_ref=kbuf.at[src],
                send_sem=send_sems.at[0, h], recv_sem=recv_sems.at[0, h],
                device_id=(right,), device_id_type=pl.DeviceIdType.MESH)
            rv = pltpu.make_async_remote_copy(
                src_ref=vbuf.at[src], dst_ref=vbuf.at[src],
                send_sem=send_sems.at[1, h], recv_sem=recv_sems.at[1, h],
                device_id=(right,), device_id_type=pl.DeviceIdType.MESH)
            rk.start()
            rv.start()
            rk.wait()
            rv.wait()

        xf = x_ref[...].reshape(B * Sq_l, D)
        qf = jnp.dot(xf, wq_ref[...], preferred_element_type=jnp.float32)

        qblocks_per_shard = Sq_l // BLK
        for b in range(B):
            ctx_heads = []
            for hq in range(Hq):
                qbh = qf[b * Sq_l:(b + 1) * Sq_l, hq * Dh:(hq + 1) * Dh]
                m = jnp.full((Sq_l, 1), -1e30, jnp.float32)
                blocks = []
                for o in range(N_DEV):
                    kc = kbuf[o, b]
                    s = lax.dot_general(
                        qbh, kc[:, hq * Dh:(hq + 1) * Dh],
                        (((1,), (1,)), ((), ())),
                        preferred_element_type=jnp.float32)
                    qi = lax.broadcasted_iota(jnp.int32, (Sq_l, Skv_l), 0)
                    ki = lax.broadcasted_iota(jnp.int32, (Sq_l, Skv_l), 1)
                    qb = qi // BLK + my * qblocks_per_shard
                    kb = ki // BLK + o * (Skv_l // BLK)
                    mask = (qb == kb) | (kb == 0) | (lax.rem(qb + kb, 3) == 0)
                    s = jnp.where(mask, s * 0.125, -1e9)
                    blocks.append(s)
                    m = jnp.maximum(m, s.max(axis=1, keepdims=True))
                ctx = jnp.zeros((Sq_l, Dh), jnp.float32)
                den = jnp.zeros((Sq_l, 1), jnp.float32)
                for o in range(N_DEV):
                    e = jnp.exp(blocks[o] - m)
                    den = den + e.sum(axis=1, keepdims=True)
                    vc = vbuf[o, b]
                    ctx = ctx + jnp.dot(e, vc[:, hq * Dh:(hq + 1) * Dh],
                                        preferred_element_type=jnp.float32)
                ctx_heads.append(ctx / den)
            ctx_b = jnp.concatenate(ctx_heads, axis=1)
            out_ref[b] = jnp.dot(ctx_b, wo_ref[...],
                                 preferred_element_type=jnp.float32)

    return pl.pallas_call(
        body,
        out_shape=jax.ShapeDtypeStruct((B, Sq_l, D), jnp.float32),
        in_specs=[pl.BlockSpec(memory_space=pltpu.VMEM)] * 5,
        out_specs=pl.BlockSpec(memory_space=pltpu.VMEM),
        scratch_shapes=[
            pltpu.VMEM((N_DEV, B, Skv_l, HD), jnp.float32),
            pltpu.VMEM((N_DEV, B, Skv_l, HD), jnp.float32),
            pltpu.SemaphoreType.DMA((2, N_DEV - 1)),
            pltpu.SemaphoreType.DMA((2, N_DEV - 1)),
        ],
        compiler_params=pltpu.CompilerParams(collective_id=0),
    )(x, Wq, k2, v2, Wo)


# device time: 76682 ns/iter; 1.4811x vs baseline; 1.4811x over previous
import jax
import jax.numpy as jnp
from jax import lax
from jax.experimental import pallas as pl
from jax.experimental.pallas import tpu as pltpu

N_DEV = 8
BLK = 64


def kernel(x, Wq, K_ext, V_ext, Wo):
    B, Sq_l, D = x.shape
    _, Skv_l, Hq, Dh = K_ext.shape
    HD = Hq * Dh

    k2 = K_ext.reshape(B, Skv_l, HD)
    v2 = V_ext.reshape(B, Skv_l, HD)

    def body(x_ref, wq_ref, k_ref, v_ref, wo_ref, out_ref,
             kbuf, vbuf, send_sems, recv_sems):
        my = lax.axis_index("i")
        left = lax.rem(my + N_DEV - 1, N_DEV)
        right = lax.rem(my + 1, N_DEV)

        barrier = pltpu.get_barrier_semaphore()
        for nbr in (left, right):
            pl.semaphore_signal(barrier, inc=1, device_id=(nbr,),
                                device_id_type=pl.DeviceIdType.MESH)
        pl.semaphore_wait(barrier, 2)

        kbuf[my] = k_ref[...]
        vbuf[my] = v_ref[...]

        def ring_copy(buf, src, t, d, h, dst_dev):
            return pltpu.make_async_remote_copy(
                src_ref=buf.at[src], dst_ref=buf.at[src],
                send_sem=send_sems.at[t, d, h], recv_sem=recv_sems.at[t, d, h],
                device_id=(dst_dev,), device_id_type=pl.DeviceIdType.MESH)

        for h in range(4):
            src_cw = lax.rem(my + N_DEV - h, N_DEV)
            rdmas = [ring_copy(kbuf, src_cw, 0, 0, h, right),
                     ring_copy(vbuf, src_cw, 1, 0, h, right)]
            if h < 3:
                src_ccw = lax.rem(my + h, N_DEV)
                rdmas += [ring_copy(kbuf, src_ccw, 0, 1, h, left),
                          ring_copy(vbuf, src_ccw, 1, 1, h, left)]
            for r in rdmas:
                r.start()
            for r in rdmas:
                r.wait()

        xf = x_ref[...].reshape(B * Sq_l, D)
        qf = jnp.dot(xf, wq_ref[...], preferred_element_type=jnp.float32)

        qblocks_per_shard = Sq_l // BLK
        for b in range(B):
            ctx_heads = []
            for hq in range(Hq):
                qbh = qf[b * Sq_l:(b + 1) * Sq_l, hq * Dh:(hq + 1) * Dh]
                m = jnp.full((Sq_l, 1), -1e30, jnp.float32)
                blocks = []
                for o in range(N_DEV):
                    kc = kbuf[o, b]
                    s = lax.dot_general(
                        qbh, kc[:, hq * Dh:(hq + 1) * Dh],
                        (((1,), (1,)), ((), ())),
                        preferred_element_type=jnp.float32)
                    qi = lax.broadcasted_iota(jnp.int32, (Sq_l, Skv_l), 0)
                    ki = lax.broadcasted_iota(jnp.int32, (Sq_l, Skv_l), 1)
                    qb = qi // BLK + my * qblocks_per_shard
                    kb = ki // BLK + o * (Skv_l // BLK)
                    mask = (qb == kb) | (kb == 0) | (lax.rem(qb + kb, 3) == 0)
                    s = jnp.where(mask, s * 0.125, -1e9)
                    blocks.append(s)
                    m = jnp.maximum(m, s.max(axis=1, keepdims=True))
                ctx = jnp.zeros((Sq_l, Dh), jnp.float32)
                den = jnp.zeros((Sq_l, 1), jnp.float32)
                for o in range(N_DEV):
                    e = jnp.exp(blocks[o] - m)
                    den = den + e.sum(axis=1, keepdims=True)
                    vc = vbuf[o, b]
                    ctx = ctx + jnp.dot(e, vc[:, hq * Dh:(hq + 1) * Dh],
                                        preferred_element_type=jnp.float32)
                ctx_heads.append(ctx / den)
            ctx_b = jnp.concatenate(ctx_heads, axis=1)
            out_ref[b] = jnp.dot(ctx_b, wo_ref[...],
                                 preferred_element_type=jnp.float32)

    return pl.pallas_call(
        body,
        out_shape=jax.ShapeDtypeStruct((B, Sq_l, D), jnp.float32),
        in_specs=[pl.BlockSpec(memory_space=pltpu.VMEM)] * 5,
        out_specs=pl.BlockSpec(memory_space=pltpu.VMEM),
        scratch_shapes=[
            pltpu.VMEM((N_DEV, B, Skv_l, HD), jnp.float32),
            pltpu.VMEM((N_DEV, B, Skv_l, HD), jnp.float32),
            pltpu.SemaphoreType.DMA((2, 2, 4)),
            pltpu.SemaphoreType.DMA((2, 2, 4)),
        ],
        compiler_params=pltpu.CompilerParams(collective_id=0),
    )(x, Wq, k2, v2, Wo)


# device time: 64168 ns/iter; 1.7700x vs baseline; 1.1950x over previous
import jax
import jax.numpy as jnp
from jax import lax
from jax.experimental import pallas as pl
from jax.experimental.pallas import tpu as pltpu

N_DEV = 8
BLK = 64


def kernel(x, Wq, K_ext, V_ext, Wo):
    B, Sq_l, D = x.shape
    _, Skv_l, Hq, Dh = K_ext.shape
    HD = Hq * Dh

    k2 = K_ext.reshape(B, Skv_l, HD)
    v2 = V_ext.reshape(B, Skv_l, HD)

    def body(x_ref, wq_ref, k_ref, v_ref, wo_ref, out_ref,
             kbuf, vbuf, send_sems, recv_sems):
        my = lax.axis_index("i")
        left = lax.rem(my + N_DEV - 1, N_DEV)
        right = lax.rem(my + 1, N_DEV)

        barrier = pltpu.get_barrier_semaphore()
        for nbr in (left, right):
            pl.semaphore_signal(barrier, inc=1, device_id=(nbr,),
                                device_id_type=pl.DeviceIdType.MESH)
        pl.semaphore_wait(barrier, 2)

        kbuf[my] = k_ref[...]
        vbuf[my] = v_ref[...]

        def ring_copy(buf, src, t, d, h, dst_dev):
            return pltpu.make_async_remote_copy(
                src_ref=buf.at[src], dst_ref=buf.at[src],
                send_sem=send_sems.at[t, d, h], recv_sem=recv_sems.at[t, d, h],
                device_id=(dst_dev,), device_id_type=pl.DeviceIdType.MESH)

        def start_hop(h):
            rdmas = [ring_copy(kbuf, lax.rem(my + N_DEV - h, N_DEV), 0, 0, h, right),
                     ring_copy(vbuf, lax.rem(my + N_DEV - h, N_DEV), 1, 0, h, right)]
            if h < 3:
                rdmas += [ring_copy(kbuf, lax.rem(my + h, N_DEV), 0, 1, h, left),
                          ring_copy(vbuf, lax.rem(my + h, N_DEV), 1, 1, h, left)]
            for r in rdmas:
                r.start()
            return rdmas

        qblocks_per_shard = Sq_l // BLK
        kblocks_per_shard = Skv_l // BLK

        state = {}
        for b in range(B):
            for hq in range(Hq):
                state[(b, hq)] = (
                    jnp.full((Sq_l, 1), -1e30, jnp.float32),
                    jnp.zeros((Sq_l, 1), jnp.float32),
                    jnp.zeros((Sq_l, Dh), jnp.float32),
                )

        def process_origin(o, qf):
            for b in range(B):
                kc = kbuf[o, b]
                vc = vbuf[o, b]
                qi = lax.broadcasted_iota(jnp.int32, (Sq_l, Skv_l), 0)
                ki = lax.broadcasted_iota(jnp.int32, (Sq_l, Skv_l), 1)
                qb = qi // BLK + my * qblocks_per_shard
                kb = ki // BLK + o * kblocks_per_shard
                mask = (qb == kb) | (kb == 0) | (lax.rem(qb + kb, 3) == 0)
                for hq in range(Hq):
                    qbh = qf[b * Sq_l:(b + 1) * Sq_l, hq * Dh:(hq + 1) * Dh]
                    s = lax.dot_general(
                        qbh, kc[:, hq * Dh:(hq + 1) * Dh],
                        (((1,), (1,)), ((), ())),
                        preferred_element_type=jnp.float32)
                    s = jnp.where(mask, s * 0.125, -1e9)
                    m_old, den, ctx = state[(b, hq)]
                    m_new = jnp.maximum(m_old, s.max(axis=1, keepdims=True))
                    scale = jnp.exp(m_old - m_new)
                    e = jnp.exp(s - m_new)
                    den = den * scale + e.sum(axis=1, keepdims=True)
                    ctx = ctx * scale + jnp.dot(
                        e, vc[:, hq * Dh:(hq + 1) * Dh],
                        preferred_element_type=jnp.float32)
                    state[(b, hq)] = (m_new, den, ctx)

        all_rdmas = []
        all_rdmas += start_hop(0)
        xf = x_ref[...].reshape(B * Sq_l, D)
        qf = jnp.dot(xf, wq_ref[...], preferred_element_type=jnp.float32)
        process_origin(my, qf)

        for h in range(4):
            for r in all_rdmas[-4 if h < 3 else -2:]:
                r.wait_recv()
            if h < 3:
                all_rdmas += start_hop(h + 1)
            process_origin(lax.rem(my + N_DEV - 1 - h, N_DEV), qf)
            if h < 3:
                process_origin(lax.rem(my + 1 + h, N_DEV), qf)

        for b in range(B):
            ctx_heads = []
            for hq in range(Hq):
                _, den, ctx = state[(b, hq)]
                ctx_heads.append(ctx / den)
            ctx_b = jnp.concatenate(ctx_heads, axis=1)
            out_ref[b] = jnp.dot(ctx_b, wo_ref[...],
                                 preferred_element_type=jnp.float32)

        for r in all_rdmas:
            r.wait_send()

    return pl.pallas_call(
        body,
        out_shape=jax.ShapeDtypeStruct((B, Sq_l, D), jnp.float32),
        in_specs=[pl.BlockSpec(memory_space=pltpu.VMEM)] * 5,
        out_specs=pl.BlockSpec(memory_space=pltpu.VMEM),
        scratch_shapes=[
            pltpu.VMEM((N_DEV, B, Skv_l, HD), jnp.float32),
            pltpu.VMEM((N_DEV, B, Skv_l, HD), jnp.float32),
            pltpu.SemaphoreType.DMA((2, 2, 4)),
            pltpu.SemaphoreType.DMA((2, 2, 4)),
        ],
        compiler_params=pltpu.CompilerParams(collective_id=0),
    )(x, Wq, k2, v2, Wo)


# device time: 56617 ns/iter; 2.0061x vs baseline; 1.1334x over previous
import jax
import jax.numpy as jnp
from jax import lax
from jax.experimental import pallas as pl
from jax.experimental.pallas import tpu as pltpu

N_DEV = 8
BLK = 64


def kernel(x, Wq, K_ext, V_ext, Wo):
    B, Sq_l, D = x.shape
    _, Skv_l, Hq, Dh = K_ext.shape
    HD = Hq * Dh

    k2 = K_ext.reshape(B, Skv_l, HD)
    v2 = V_ext.reshape(B, Skv_l, HD)

    def body(x_ref, wq_ref, k_ref, v_ref, wo_ref, out_ref,
             kbuf, vbuf, send_sems, recv_sems):
        my = lax.axis_index("i")

        def r2m(p):
            return jnp.where(p < 4, p, 11 - p)

        r = r2m(my)
        right = r2m(lax.rem(r + 1, N_DEV))
        left = r2m(lax.rem(r + N_DEV - 1, N_DEV))

        barrier = pltpu.get_barrier_semaphore()
        for nbr in (left, right):
            pl.semaphore_signal(barrier, inc=1, device_id=(nbr,),
                                device_id_type=pl.DeviceIdType.MESH)
        pl.semaphore_wait(barrier, 2)

        kbuf[my] = k_ref[...]
        vbuf[my] = v_ref[...]

        def copy(src_slice, t, d, h, dst_dev):
            buf = (kbuf, vbuf)[t]
            ref = buf.at[src_slice] if not isinstance(src_slice, tuple) \
                else buf.at[src_slice[0], src_slice[1]]
            return pltpu.make_async_remote_copy(
                src_ref=ref, dst_ref=ref,
                send_sem=send_sems.at[t, d, h], recv_sem=recv_sems.at[t, d, h],
                device_id=(dst_dev,), device_id_type=pl.DeviceIdType.MESH)

        def start_hop(h):
            src_cw = r2m(lax.rem(r + N_DEV - h, N_DEV))
            src_ccw = r2m(lax.rem(r + h, N_DEV))
            if h < 3:
                rdmas = [copy(src_cw, 0, 0, h, right),
                         copy(src_cw, 1, 0, h, right),
                         copy(src_ccw, 0, 1, h, left),
                         copy(src_ccw, 1, 1, h, left)]
            else:
                rdmas = [copy((src_cw, 0), 0, 0, h, right),
                         copy((src_cw, 0), 1, 0, h, right),
                         copy((src_ccw, 1), 0, 1, h, left),
                         copy((src_ccw, 1), 1, 1, h, left)]
            for rd in rdmas:
                rd.start()
            return rdmas

        qblocks_per_shard = Sq_l // BLK
        kblocks_per_shard = Skv_l // BLK

        state = {}
        for b in range(B):
            for hq in range(Hq):
                state[(b, hq)] = (
                    jnp.full((Sq_l, 1), -1e30, jnp.float32),
                    jnp.zeros((Sq_l, 1), jnp.float32),
                    jnp.zeros((Sq_l, Dh), jnp.float32),
                )

        def process_origin(o, qf):
            for b in range(B):
                kc = kbuf[o, b]
                vc = vbuf[o, b]
                qi = lax.broadcasted_iota(jnp.int32, (Sq_l, Skv_l), 0)
                ki = lax.broadcasted_iota(jnp.int32, (Sq_l, Skv_l), 1)
                qb = qi // BLK + my * qblocks_per_shard
                kb = ki // BLK + o * kblocks_per_shard
                mask = (qb == kb) | (kb == 0) | (lax.rem(qb + kb, 3) == 0)
                for hq in range(Hq):
                    qbh = qf[b * Sq_l:(b + 1) * Sq_l, hq * Dh:(hq + 1) * Dh]
                    s = lax.dot_general(
                        qbh, kc[:, hq * Dh:(hq + 1) * Dh],
                        (((1,), (1,)), ((), ())),
                        preferred_element_type=jnp.float32)
                    s = jnp.where(mask, s * 0.125, -1e9)
                    m_old, den, ctx = state[(b, hq)]
                    m_new = jnp.maximum(m_old, s.max(axis=1, keepdims=True))
                    scale = jnp.exp(m_old - m_new)
                    e = jnp.exp(s - m_new)
                    den = den * scale + e.sum(axis=1, keepdims=True)
                    ctx = ctx * scale + jnp.dot(
                        e, vc[:, hq * Dh:(hq + 1) * Dh],
                        preferred_element_type=jnp.float32)
                    state[(b, hq)] = (m_new, den, ctx)

        all_rdmas = []
        all_rdmas += start_hop(0)
        xf = x_ref[...].reshape(B * Sq_l, D)
        qf = jnp.dot(xf, wq_ref[...], preferred_element_type=jnp.float32)
        process_origin(my, qf)

        for h in range(4):
            for rd in all_rdmas[-4:]:
                rd.wait_recv()
            if h < 3:
                all_rdmas += start_hop(h + 1)
            if h < 3:
                process_origin(r2m(lax.rem(r + N_DEV - 1 - h, N_DEV)), qf)
                process_origin(r2m(lax.rem(r + 1 + h, N_DEV)), qf)
            else:
                process_origin(r2m(lax.rem(r + 4, N_DEV)), qf)

        for b in range(B):
            ctx_heads = []
            for hq in range(Hq):
                _, den, ctx = state[(b, hq)]
                ctx_heads.append(ctx / den)
            ctx_b = jnp.concatenate(ctx_heads, axis=1)
            out_ref[b] = jnp.dot(ctx_b, wo_ref[...],
                                 preferred_element_type=jnp.float32)

        for rd in all_rdmas:
            rd.wait_send()

    return pl.pallas_call(
        body,
        out_shape=jax.ShapeDtypeStruct((B, Sq_l, D), jnp.float32),
        in_specs=[pl.BlockSpec(memory_space=pltpu.VMEM)] * 5,
        out_specs=pl.BlockSpec(memory_space=pltpu.VMEM),
        scratch_shapes=[
            pltpu.VMEM((N_DEV, B, Skv_l, HD), jnp.float32),
            pltpu.VMEM((N_DEV, B, Skv_l, HD), jnp.float32),
            pltpu.SemaphoreType.DMA((2, 2, 4)),
            pltpu.SemaphoreType.DMA((2, 2, 4)),
        ],
        compiler_params=pltpu.CompilerParams(collective_id=0),
    )(x, Wq, k2, v2, Wo)
